# baseline (speedup 1.0000x reference)
import dataclasses

import jax
import jax.numpy as jnp
from jax.experimental import pallas as pl
from jax.experimental.pallas import tpu as pltpu
from jax.experimental.pallas import tpu_sc as plsc

N = 4194304
F = 16
LANES = 16
BLOCK = 2048


def _sc_bwb(gs0, a1, A, rh, FGs):
    mesh = plsc.VectorSubcoreMesh(core_axis_name="core",
                                  subcore_axis_name="subcore")
    cp = pltpu.CompilerParams()
    if "needs_layout_passes" in pltpu.CompilerParams.__dataclass_fields__:
        cp = dataclasses.replace(cp, needs_layout_passes=False)

    @pl.kernel(
        out_type=jax.ShapeDtypeStruct((N,), jnp.float32),
        mesh=mesh,
        compiler_params=cp,
        scratch_types=[pltpu.VMEM((F,), jnp.float32),
                       pltpu.VMEM((F,), jnp.float32)],
    )
    def kernel(gs0_hbm, a1_hbm, a_hbm, rh_hbm, fgs_hbm, out_hbm,
               gs0_v, a1_v):
        pltpu.sync_copy(gs0_hbm, gs0_v)
        pltpu.sync_copy(a1_hbm, a1_v)

        def body(a_vmem, rh_vmem, fgs_vmem, out_vmem):
            @pl.loop(0, BLOCK, step=LANES)
            def _(c):
                sl = pl.ds(c, LANES)
                idx = fgs_vmem[sl]
                g = plsc.load_gather(gs0_v, [idx])
                a = plsc.load_gather(a1_v, [idx])
                out_vmem[sl] = g + a * a_vmem[sl] * rh_vmem[sl] * (1.0 / 420.0)

        pltpu.emit_pipeline(
            body,
            grid=(N // BLOCK,),
            in_specs=[pl.BlockSpec((BLOCK,), lambda i: (i,))] * 3,
            out_specs=[pl.BlockSpec((BLOCK,), lambda i: (i,))],
            core_axis_name=("core", "subcore"),
            dimension_semantics=(pltpu.PARALLEL,),
        )(a_hbm, rh_hbm, fgs_hbm, out_hbm)

    return kernel(gs0, a1, A, rh, FGs)


def kernel(gs0, a1, A, rh, FGs):
    return _sc_bwb(gs0, a1, A, rh, FGs)


# speedup vs baseline: 2.5117x; 2.5117x over previous
import dataclasses

import jax
import jax.numpy as jnp
from jax.experimental import pallas as pl
from jax.experimental.pallas import tpu as pltpu
from jax.experimental.pallas import tpu_sc as plsc

N = 4194304
F = 16
LANES = 16
BLOCK = 4096


def _sc_bwb(gs0, a1, A, rh, FGs):
    mesh = plsc.VectorSubcoreMesh(core_axis_name="core",
                                  subcore_axis_name="subcore")
    cp = pltpu.CompilerParams()
    if "needs_layout_passes" in pltpu.CompilerParams.__dataclass_fields__:
        cp = dataclasses.replace(cp, needs_layout_passes=False)

    @pl.kernel(
        out_type=jax.ShapeDtypeStruct((N,), jnp.float32),
        mesh=mesh,
        compiler_params=cp,
        scratch_types=[pltpu.VMEM((F,), jnp.float32),
                       pltpu.VMEM((F,), jnp.float32)],
    )
    def kernel(gs0_hbm, a1_hbm, a_hbm, rh_hbm, fgs_hbm, out_hbm,
               gs0_v, a1_v):
        pltpu.sync_copy(gs0_hbm, gs0_v)
        pltpu.sync_copy(a1_hbm, a1_v)

        def body(a_vmem, rh_vmem, fgs_vmem, out_vmem):
            @plsc.parallel_loop(0, BLOCK, LANES, unroll=8)
            def _(c):
                sl = pl.ds(c, LANES)
                idx = fgs_vmem[sl]
                g = plsc.load_gather(gs0_v, [idx])
                a = plsc.load_gather(a1_v, [idx])
                out_vmem[sl] = g + a * a_vmem[sl] * rh_vmem[sl] * (1.0 / 420.0)

        pltpu.emit_pipeline(
            body,
            grid=(N // BLOCK,),
            in_specs=[pl.BlockSpec((BLOCK,), lambda i: (i,))] * 3,
            out_specs=[pl.BlockSpec((BLOCK,), lambda i: (i,))],
            core_axis_name=("core", "subcore"),
            dimension_semantics=(pltpu.PARALLEL,),
        )(a_hbm, rh_hbm, fgs_hbm, out_hbm)

    return kernel(gs0, a1, A, rh, FGs)


def kernel(gs0, a1, A, rh, FGs):
    return _sc_bwb(gs0, a1, A, rh, FGs)


# speedup vs baseline: 2.7281x; 1.0861x over previous
import dataclasses

import jax
import jax.numpy as jnp
from jax.experimental import pallas as pl
from jax.experimental.pallas import tpu as pltpu
from jax.experimental.pallas import tpu_sc as plsc

N = 4194304
F = 16
LANES = 16
BLOCK = 8192


def _sc_bwb(gs0, a1, A, rh, FGs):
    mesh = plsc.VectorSubcoreMesh(core_axis_name="core",
                                  subcore_axis_name="subcore")
    cp = pltpu.CompilerParams()
    if "needs_layout_passes" in pltpu.CompilerParams.__dataclass_fields__:
        cp = dataclasses.replace(cp, needs_layout_passes=False)

    @pl.kernel(
        out_type=jax.ShapeDtypeStruct((N,), jnp.float32),
        mesh=mesh,
        compiler_params=cp,
        scratch_types=[pltpu.VMEM((F,), jnp.float32),
                       pltpu.VMEM((F,), jnp.float32)],
    )
    def kernel(gs0_hbm, a1_hbm, a_hbm, rh_hbm, fgs_hbm, out_hbm,
               gs0_v, a1_v):
        pltpu.sync_copy(gs0_hbm, gs0_v)
        pltpu.sync_copy(a1_hbm, a1_v)

        def body(a_vmem, rh_vmem, fgs_vmem, out_vmem):
            @plsc.parallel_loop(0, BLOCK, LANES, unroll=8)
            def _(c):
                sl = pl.ds(c, LANES)
                idx = fgs_vmem[sl]
                g = plsc.load_gather(gs0_v, [idx])
                a = plsc.load_gather(a1_v, [idx])
                out_vmem[sl] = g + a * a_vmem[sl] * rh_vmem[sl] * (1.0 / 420.0)

        pltpu.emit_pipeline(
            body,
            grid=(N // BLOCK,),
            in_specs=[pl.BlockSpec((BLOCK,), lambda i: (i,))] * 3,
            out_specs=[pl.BlockSpec((BLOCK,), lambda i: (i,))],
            core_axis_name=("core", "subcore"),
            dimension_semantics=(pltpu.PARALLEL,),
        )(a_hbm, rh_hbm, fgs_hbm, out_hbm)

    return kernel(gs0, a1, A, rh, FGs)


def kernel(gs0, a1, A, rh, FGs):
    return _sc_bwb(gs0, a1, A, rh, FGs)
